# baseline (speedup 1.0000x reference)
import functools

import jax
import jax.numpy as jnp
from jax import lax
from jax.experimental import pallas as pl
from jax.experimental.pallas import tpu as pltpu
from jax.experimental.pallas import tpu_sc as plsc

_VOCAB = 33
_DIM = 1280
_BATCH = 4
_SEQ = 2048
_B_TOTAL = _BATCH * _SEQ
_NUM_WORKERS = 32
_B_PER_W = _B_TOTAL // _NUM_WORKERS
_CHUNK = 16
_NCHUNK = _B_PER_W // _CHUNK
_LANES = 16
_VPR = _DIM // _LANES


def _body(table_hbm, idx_hbm, out_hbm, table_v, idx_v, rows0, rows1,
          wsem0, wsem1):
    wid = lax.axis_index("s") * 2 + lax.axis_index("c")
    base = wid * _B_PER_W
    rows = (rows0, rows1)
    wsem = (wsem0, wsem1)

    pltpu.sync_copy(table_hbm, table_v)
    pltpu.sync_copy(idx_hbm.at[pl.ds(base, _B_PER_W)], idx_v)

    def do_chunk(c, buf, sem, first):
        @pl.when(jnp.logical_not(first))
        def _():
            pltpu.make_async_copy(
                buf, out_hbm.at[pl.ds(0, _CHUNK * _DIM)], sem).wait()
        ids = idx_v[pl.ds(c * _CHUNK, _LANES)]
        for lane in range(_LANES):
            src = ids[lane] * _DIM
            dst = lane * _DIM
            grp = 8
            vals = [table_v[pl.ds(src + k * _LANES, _LANES)]
                    for k in range(grp)]
            for k0 in range(grp, _VPR, grp):
                nxt = []
                for k in range(grp):
                    nxt.append(table_v[pl.ds(src + (k0 + k) * _LANES,
                                             _LANES)])
                    buf[pl.ds(dst + (k0 - grp + k) * _LANES, _LANES)] = (
                        vals[k])
                vals = nxt
            for k in range(grp):
                buf[pl.ds(dst + (_VPR - grp + k) * _LANES, _LANES)] = vals[k]
        pltpu.async_copy(
            buf,
            out_hbm.at[pl.ds((base + c * _CHUNK) * _DIM, _CHUNK * _DIM)],
            sem)

    def pair(p, _):
        do_chunk(2 * p, rows[0], wsem[0], p == 0)
        do_chunk(2 * p + 1, rows[1], wsem[1], p == 0)
        return 0

    lax.fori_loop(0, _NCHUNK // 2, pair, 0)
    for k in range(2):
        pltpu.make_async_copy(
            rows[k], out_hbm.at[pl.ds(0, _CHUNK * _DIM)], wsem[k]).wait()


_gather = functools.partial(
    pl.kernel,
    out_type=jax.ShapeDtypeStruct((_B_TOTAL * _DIM,), jnp.float32),
    mesh=plsc.VectorSubcoreMesh(core_axis_name="c", subcore_axis_name="s"),
    scratch_types=[
        pltpu.VMEM((_VOCAB * _DIM,), jnp.float32),
        pltpu.VMEM((_B_PER_W,), jnp.int32),
        pltpu.VMEM((_CHUNK * _DIM,), jnp.float32),
        pltpu.VMEM((_CHUNK * _DIM,), jnp.float32),
        pltpu.SemaphoreType.DMA,
        pltpu.SemaphoreType.DMA,
    ],
)(_body)


@jax.jit
def kernel(input_ids, weight):
    ids = input_ids.reshape(-1).astype(jnp.int32)
    table = weight.reshape(-1).astype(jnp.float32)
    out = _gather(table, ids)
    return out.reshape(_BATCH, _SEQ, _DIM).astype(weight.dtype)


# speedup vs baseline: 1.6074x; 1.6074x over previous
import functools

import jax
import jax.numpy as jnp
from jax import lax
from jax.experimental import pallas as pl
from jax.experimental.pallas import tpu as pltpu
from jax.experimental.pallas import tpu_sc as plsc

_VOCAB = 33
_DIM = 1280
_BATCH = 4
_SEQ = 2048
_B_TOTAL = _BATCH * _SEQ
_NUM_WORKERS = 32
_B_PER_W = _B_TOTAL // _NUM_WORKERS
_W_PER_ROW = _SEQ // _B_PER_W
_CHUNK = 16
_NCHUNK = _B_PER_W // _CHUNK
_LANES = 16
_VPR = _DIM // _LANES


def _body(table_hbm, idx_hbm, out_hbm, table_v, idx_v, rows0, rows1,
          wsem0, wsem1):
    wid = lax.axis_index("s") * 2 + lax.axis_index("c")
    base = wid * _B_PER_W
    rows = (rows0, rows1)
    wsem = (wsem0, wsem1)

    pltpu.sync_copy(table_hbm, table_v)
    pltpu.sync_copy(
        idx_hbm.at[wid // _W_PER_ROW,
                   pl.ds((wid % _W_PER_ROW) * _B_PER_W, _B_PER_W)],
        idx_v)

    def copy_row(i, lane, buf):
        grp = 8
        vals = [table_v[i, pl.ds(k * _LANES, _LANES)] for k in range(grp)]
        for k0 in range(grp, _VPR, grp):
            nxt = []
            for k in range(grp):
                nxt.append(table_v[i, pl.ds((k0 + k) * _LANES, _LANES)])
                buf[lane, pl.ds((k0 - grp + k) * _LANES, _LANES)] = vals[k]
            vals = nxt
        for k in range(grp):
            buf[lane, pl.ds((_VPR - grp + k) * _LANES, _LANES)] = vals[k]

    def do_chunk(c, buf, sem, first):
        @pl.when(jnp.logical_not(first))
        def _():
            pltpu.make_async_copy(
                buf, out_hbm.at[pl.ds(0, _CHUNK)], sem).wait()
        ids = idx_v[pl.ds(c * _CHUNK, _LANES)]
        for lane in range(_LANES):
            copy_row(ids[lane], lane, buf)
        pltpu.async_copy(
            buf, out_hbm.at[pl.ds(base + c * _CHUNK, _CHUNK)], sem)

    def pair(p, _):
        do_chunk(2 * p, rows[0], wsem[0], p == 0)
        do_chunk(2 * p + 1, rows[1], wsem[1], p == 0)
        return 0

    lax.fori_loop(0, _NCHUNK // 2, pair, 0)
    for k in range(2):
        pltpu.make_async_copy(
            rows[k], out_hbm.at[pl.ds(0, _CHUNK)], wsem[k]).wait()


_gather = functools.partial(
    pl.kernel,
    out_type=jax.ShapeDtypeStruct((_B_TOTAL, _DIM), jnp.float32),
    mesh=plsc.VectorSubcoreMesh(core_axis_name="c", subcore_axis_name="s"),
    scratch_types=[
        pltpu.VMEM((_VOCAB, _DIM), jnp.float32),
        pltpu.VMEM((_B_PER_W,), jnp.int32),
        pltpu.VMEM((_CHUNK, _DIM), jnp.float32),
        pltpu.VMEM((_CHUNK, _DIM), jnp.float32),
        pltpu.SemaphoreType.DMA,
        pltpu.SemaphoreType.DMA,
    ],
)(_body)


@jax.jit
def kernel(input_ids, weight):
    out = _gather(weight.astype(jnp.float32), input_ids.astype(jnp.int32))
    return out.reshape(_BATCH, _SEQ, _DIM).astype(weight.dtype)


# speedup vs baseline: 2.6932x; 1.6755x over previous
import functools

import jax
import jax.numpy as jnp
from jax import lax
from jax.experimental import pallas as pl
from jax.experimental.pallas import tpu as pltpu
from jax.experimental.pallas import tpu_sc as plsc

_VOCAB = 33
_DIM = 1280
_BATCH = 4
_SEQ = 2048
_B_TOTAL = _BATCH * _SEQ
_NUM_WORKERS = 32
_B_PER_W = _B_TOTAL // _NUM_WORKERS
_W_PER_ROW = _SEQ // _B_PER_W
_LANES = 16
_NCHUNK = _B_PER_W // _LANES


def _body(table_hbm, idx_hbm, out_hbm, table_v, idx_v, sem0, sem1):
    wid = lax.axis_index("s") * 2 + lax.axis_index("c")
    base = wid * _B_PER_W
    sems = (sem0, sem1)

    pltpu.sync_copy(table_hbm, table_v)
    pltpu.sync_copy(
        idx_hbm.at[wid // _W_PER_ROW,
                   pl.ds((wid % _W_PER_ROW) * _B_PER_W, _B_PER_W)],
        idx_v)

    def fire(c, sem):
        ids = idx_v[pl.ds(c * _LANES, _LANES)]
        for lane in range(_LANES):
            pltpu.async_copy(
                table_v.at[ids[lane]],
                out_hbm.at[base + c * _LANES + lane],
                sem)

    def drain(sem):
        for _ in range(_LANES):
            pltpu.make_async_copy(
                table_v.at[0], out_hbm.at[0], sem).wait()

    def pair(p, _):
        @pl.when(p > 0)
        def _():
            drain(sems[0])
        fire(2 * p, sems[0])

        @pl.when(p > 0)
        def _():
            drain(sems[1])
        fire(2 * p + 1, sems[1])
        return 0

    lax.fori_loop(0, _NCHUNK // 2, pair, 0)
    drain(sems[0])
    drain(sems[1])


_gather = functools.partial(
    pl.kernel,
    out_type=jax.ShapeDtypeStruct((_B_TOTAL, _DIM), jnp.float32),
    mesh=plsc.VectorSubcoreMesh(core_axis_name="c", subcore_axis_name="s"),
    scratch_types=[
        pltpu.VMEM((_VOCAB, _DIM), jnp.float32),
        pltpu.VMEM((_B_PER_W,), jnp.int32),
        pltpu.SemaphoreType.DMA,
        pltpu.SemaphoreType.DMA,
    ],
)(_body)


@jax.jit
def kernel(input_ids, weight):
    out = _gather(weight.astype(jnp.float32), input_ids.astype(jnp.int32))
    return out.reshape(_BATCH, _SEQ, _DIM).astype(weight.dtype)
